# baseline (speedup 1.0000x reference)
import functools

import jax
import jax.numpy as jnp
from jax import lax
from jax.experimental import pallas as pl
from jax.experimental.pallas import tpu as pltpu
from jax.experimental.pallas import tpu_sc as plsc

NC = 2
NS = 16
NW = NC * NS
L = 16

S = 8
SCS = 3
TCS = S - SCS
B = 4096
C = 1000
RT = S * C
CPW = B // NW
G = CPW // L
CHUNK = 200
CPS = C // CHUNK
NCHUNKS = SCS * CPS
BBLK = 512
NBB = B // BBLK
NBUF = 3
assert NCHUNKS % NBUF == 0
RB = CHUNK // 8

_mesh = plsc.VectorSubcoreMesh(
    core_axis_name="c", subcore_axis_name="s", num_cores=NC, num_subcores=NS
)


@functools.partial(
    pl.kernel,
    out_type=jax.ShapeDtypeStruct((NW, L), jnp.float32),
    mesh=_mesh,
    compiler_params=pltpu.CompilerParams(needs_layout_passes=False),
    scratch_types=[
        pltpu.VMEM((CHUNK, CPW), jnp.float32),
        pltpu.VMEM((CHUNK, CPW), jnp.float32),
        pltpu.VMEM((CHUNK, CPW), jnp.float32),
        pltpu.VMEM((CPW,), jnp.int32),
        pltpu.VMEM((L,), jnp.float32),
        pltpu.SemaphoreType.DMA,
        pltpu.SemaphoreType.DMA,
        pltpu.SemaphoreType.DMA,
    ],
)
def _loss_partials(
    pred_hbm, label_hbm, out_hbm, buf0, buf1, buf2, lbl, res,
    sem0, sem1, sem2,
):
    cid = lax.axis_index("c")
    sid = lax.axis_index("s")
    wid = sid * NC + cid
    col0 = wid * CPW
    pltpu.sync_copy(label_hbm.at[pl.ds(col0, CPW)], lbl)

    lane = lax.iota(jnp.int32, L)
    ninf = jnp.full((L,), -jnp.inf, jnp.float32)
    zero = jnp.zeros((L,), jnp.float32)
    sems = (sem0, sem1, sem2)
    bufs = (buf0, buf1, buf2)

    def chunk_src(c):
        return pred_hbm.at[pl.ds(c * CHUNK, CHUNK), pl.ds(col0, CPW)]

    for b in range(NBUF):
        pltpu.async_copy(chunk_src(b), bufs[b], sems[b])

    def ring_body(g2, carry):
        acc, ms, fys = carry
        for b in range(NBUF):
            c = g2 * NBUF + b
            pltpu.make_async_copy(chunk_src(c), bufs[b], sems[b]).wait()
            buf = bufs[b]
            base = lax.rem(c, CPS) * CHUNK

            ms = list(ms)
            fys = list(fys)
            for g in range(G):
                lblg = lbl[pl.ds(g * L, L)]
                li = lblg - base
                valid = (li >= 0) & (li < CHUNK)
                lic = jnp.clip(li, 0, CHUNK - 1)
                colg = lane + g * L
                got = plsc.load_gather(buf, [lic, colg], mask=valid)
                fys[g] = jnp.where(valid, got, fys[g])
                plsc.store_scatter(
                    buf, [lic, colg],
                    jnp.full((L,), -1e10, jnp.float32), mask=valid,
                )

            def rb_body(rb, ms_t):
                out = list(ms_t)
                for rr in range(8):
                    r = rb * 8 + rr
                    for g in range(G):
                        out[g] = jnp.maximum(out[g], buf[r, pl.ds(g * L, L)])
                return tuple(out)

            ms = lax.fori_loop(0, RB, rb_body, tuple(ms))

            done = jnp.broadcast_to(lax.rem(c, CPS) == CPS - 1, (L,))
            loss_sum = zero
            for g in range(G):
                loss_sum = (
                    loss_sum
                    + jnp.maximum(2.0 - fys[g], 0.0)
                    + jnp.maximum(1.0 + ms[g], 0.0)
                )
            acc = jnp.where(done, acc + loss_sum, acc)
            ms = tuple(jnp.where(done, ninf, m) for m in ms)
            fys = tuple(jnp.where(done, zero, f) for f in fys)

            @pl.when(c + NBUF < NCHUNKS)
            def _():
                pltpu.async_copy(chunk_src(c + NBUF), bufs[b], sems[b])

        return acc, ms, fys

    init = (zero, (ninf,) * G, (zero,) * G)
    acc, _, _ = lax.fori_loop(0, NCHUNKS // NBUF, ring_body, init)
    res[...] = acc
    pltpu.sync_copy(res, out_hbm.at[wid])


def _tc_body(lbl_ref, x_ref, o_ref):
    s = pl.program_id(0)
    bb = pl.program_id(1)
    x = x_ref[...]
    lbl = lbl_ref[pl.ds(bb * BBLK, BBLK)]
    cls = lax.broadcasted_iota(jnp.int32, (C, BBLK), 0)
    m = cls == lbl[None, :]
    xm = jnp.where(m, jnp.float32(-1e10), x)
    colmax = jnp.max(xm, axis=0)
    fy = jnp.max(jnp.where(m, x, -jnp.inf), axis=0)
    loss = jnp.maximum(2.0 - fy, 0.0) + jnp.maximum(1.0 + colmax, 0.0)
    ps = jnp.sum(loss)

    @pl.when((s == 0) & (bb == 0))
    def _():
        o_ref[...] = jnp.zeros((1, 1), jnp.float32)

    o_ref[...] += ps.reshape(1, 1)


_tc_loss = pl.pallas_call(
    _tc_body,
    grid=(TCS, NBB),
    in_specs=[
        pl.BlockSpec((B,), lambda s, bb: (0,)),
        pl.BlockSpec((C, BBLK), lambda s, bb: (SCS + s, bb)),
    ],
    out_specs=pl.BlockSpec((1, 1), lambda s, bb: (0, 0)),
    out_shape=jax.ShapeDtypeStruct((1, 1), jnp.float32),
)


def kernel(prediction, label):
    pred_t = prediction.reshape(S, B, C).transpose(0, 2, 1).reshape(RT, B)
    partials = _loss_partials(pred_t, label)
    tc_sum = _tc_loss(label, pred_t)
    return (jnp.sum(partials) + tc_sum[0, 0]) / jnp.float32(S * B)


# speedup vs baseline: 1.0838x; 1.0838x over previous
import functools

import jax
import jax.numpy as jnp
from jax import lax
from jax.experimental import pallas as pl
from jax.experimental.pallas import tpu as pltpu
from jax.experimental.pallas import tpu_sc as plsc

NC = 2
NS = 16
NW = NC * NS
L = 16

S = 8
SCS = 4
TCS = S - SCS
B = 4096
C = 1000
RT = S * C
CPW = B // NW
G = CPW // L
CHUNK = 200
CPS = C // CHUNK
NCHUNKS = SCS * CPS
BBLK = 512
NBB = B // BBLK
NBUF = 4
assert NCHUNKS % NBUF == 0
RB = CHUNK // 8

_mesh = plsc.VectorSubcoreMesh(
    core_axis_name="c", subcore_axis_name="s", num_cores=NC, num_subcores=NS
)


@functools.partial(
    pl.kernel,
    out_type=jax.ShapeDtypeStruct((NW, L), jnp.float32),
    mesh=_mesh,
    compiler_params=pltpu.CompilerParams(needs_layout_passes=False),
    scratch_types=[
        pltpu.VMEM((CHUNK, CPW), jnp.float32),
        pltpu.VMEM((CHUNK, CPW), jnp.float32),
        pltpu.VMEM((CHUNK, CPW), jnp.float32),
        pltpu.VMEM((CHUNK, CPW), jnp.float32),
        pltpu.VMEM((CPW,), jnp.int32),
        pltpu.VMEM((L,), jnp.float32),
        pltpu.SemaphoreType.DMA,
        pltpu.SemaphoreType.DMA,
        pltpu.SemaphoreType.DMA,
        pltpu.SemaphoreType.DMA,
    ],
)
def _loss_partials(
    pred_hbm, label_hbm, out_hbm, buf0, buf1, buf2, buf3, lbl, res,
    sem0, sem1, sem2, sem3,
):
    cid = lax.axis_index("c")
    sid = lax.axis_index("s")
    wid = sid * NC + cid
    col0 = wid * CPW
    pltpu.sync_copy(label_hbm.at[pl.ds(col0, CPW)], lbl)

    lane = lax.iota(jnp.int32, L)
    ninf = jnp.full((L,), -jnp.inf, jnp.float32)
    zero = jnp.zeros((L,), jnp.float32)
    sems = (sem0, sem1, sem2, sem3)
    bufs = (buf0, buf1, buf2, buf3)

    def chunk_src(c):
        return pred_hbm.at[pl.ds(c * CHUNK, CHUNK), pl.ds(col0, CPW)]

    for b in range(NBUF):
        pltpu.async_copy(chunk_src(b), bufs[b], sems[b])

    def ring_body(g2, carry):
        acc, ms, fys = carry
        for b in range(NBUF):
            c = g2 * NBUF + b
            pltpu.make_async_copy(chunk_src(c), bufs[b], sems[b]).wait()
            buf = bufs[b]
            base = lax.rem(c, CPS) * CHUNK

            ms = list(ms)
            fys = list(fys)
            for g in range(G):
                lblg = lbl[pl.ds(g * L, L)]
                li = lblg - base
                valid = (li >= 0) & (li < CHUNK)
                lic = jnp.clip(li, 0, CHUNK - 1)
                colg = lane + g * L
                got = plsc.load_gather(buf, [lic, colg], mask=valid)
                fys[g] = jnp.where(valid, got, fys[g])
                plsc.store_scatter(
                    buf, [lic, colg],
                    jnp.full((L,), -1e10, jnp.float32), mask=valid,
                )

            def rb_body(rb, ms_t):
                out = list(ms_t)
                for rr in range(8):
                    r = rb * 8 + rr
                    for g in range(G):
                        out[g] = jnp.maximum(out[g], buf[r, pl.ds(g * L, L)])
                return tuple(out)

            ms = lax.fori_loop(0, RB, rb_body, tuple(ms))

            done = jnp.broadcast_to(lax.rem(c, CPS) == CPS - 1, (L,))
            loss_sum = zero
            for g in range(G):
                loss_sum = (
                    loss_sum
                    + jnp.maximum(2.0 - fys[g], 0.0)
                    + jnp.maximum(1.0 + ms[g], 0.0)
                )
            acc = jnp.where(done, acc + loss_sum, acc)
            ms = tuple(jnp.where(done, ninf, m) for m in ms)
            fys = tuple(jnp.where(done, zero, f) for f in fys)

            @pl.when(c + NBUF < NCHUNKS)
            def _():
                pltpu.async_copy(chunk_src(c + NBUF), bufs[b], sems[b])

        return acc, ms, fys

    init = (zero, (ninf,) * G, (zero,) * G)
    acc, _, _ = lax.fori_loop(0, NCHUNKS // NBUF, ring_body, init)
    res[...] = acc
    pltpu.sync_copy(res, out_hbm.at[wid])


def _tc_body(lbl_ref, x_ref, o_ref):
    s = pl.program_id(0)
    bb = pl.program_id(1)
    x = x_ref[...]
    lbl = lbl_ref[pl.ds(bb * BBLK, BBLK)]
    cls = lax.broadcasted_iota(jnp.int32, (C, BBLK), 0)
    m = cls == lbl[None, :]
    xm = jnp.where(m, jnp.float32(-1e10), x)
    colmax = jnp.max(xm, axis=0)
    fy = jnp.max(jnp.where(m, x, -jnp.inf), axis=0)
    loss = jnp.maximum(2.0 - fy, 0.0) + jnp.maximum(1.0 + colmax, 0.0)
    ps = jnp.sum(loss)

    @pl.when((s == 0) & (bb == 0))
    def _():
        o_ref[...] = jnp.zeros((1, 1), jnp.float32)

    o_ref[...] += ps.reshape(1, 1)


_tc_loss = pl.pallas_call(
    _tc_body,
    grid=(TCS, NBB),
    in_specs=[
        pl.BlockSpec((B,), lambda s, bb: (0,)),
        pl.BlockSpec((C, BBLK), lambda s, bb: (SCS + s, bb)),
    ],
    out_specs=pl.BlockSpec((1, 1), lambda s, bb: (0, 0)),
    out_shape=jax.ShapeDtypeStruct((1, 1), jnp.float32),
)


def kernel(prediction, label):
    pred_t = prediction.reshape(S, B, C).transpose(0, 2, 1).reshape(RT, B)
    partials = _loss_partials(pred_t, label)
    tc_sum = _tc_loss(label, pred_t)
    return (jnp.sum(partials) + tc_sum[0, 0]) / jnp.float32(S * B)
